# baseline (speedup 1.0000x reference)
import functools

import jax
import jax.numpy as jnp
from jax import lax
from jax.experimental import pallas as pl
from jax.experimental.pallas import tpu as pltpu
from jax.experimental.pallas import tpu_sc as plsc

B, L, D = 2, 512, 64
_MAXP = (128, 395, 52)
_ROWS = (257, 791, 105)
_RPAD = (264, 792, 112)


def _proj_body(em, wm, bm, ec, wc, bc, ea, wa, ba, om, oc, oa):
    dn = (((1,), (1,)), ((), ()))
    om[...] = lax.dot_general(em[...], wm[...], dn,
                              preferred_element_type=jnp.float32) + bm[...]
    oc[...] = lax.dot_general(ec[...], wc[...], dn,
                              preferred_element_type=jnp.float32) + bc[...]
    oa[...] = lax.dot_general(ea[...], wa[...], dn,
                              preferred_element_type=jnp.float32) + ba[...]


def _make_sc_kernel():
    info = plsc.get_sparse_core_info()
    nc, ns = info.num_cores, info.num_subcores
    nw = nc * ns
    rows_per_phase = (B * L) // nw
    mesh = plsc.VectorSubcoreMesh(core_axis_name="c", subcore_axis_name="s")

    out_type = [jax.ShapeDtypeStruct((B, L, 8, 4, 8, 128), jnp.float32)
                for _ in range(3)]
    scratch = [
        pltpu.VMEM((3 * B * L,), jnp.int32),
        pltpu.VMEM((L,), jnp.int32),
        pltpu.VMEM((2, 4, 4, 8, 128), jnp.float32),
        pltpu.VMEM((_RPAD[0] * D,), jnp.float32),
        pltpu.VMEM((_RPAD[1] * D,), jnp.float32),
        pltpu.VMEM((_RPAD[2] * D,), jnp.float32),
        pltpu.SemaphoreType.DMA,
        pltpu.SemaphoreType.DMA,
    ]

    @functools.partial(
        pl.kernel, mesh=mesh, out_type=out_type, scratch_types=scratch,
        compiler_params=pltpu.CompilerParams(needs_layout_passes=False,
                                             use_tc_tiling_on_sc=False))
    def sc(pm, pc, pa, im, ic, ia, om, oc, oa,
           ids_v, pos_v, stage_v, tm_v, tc_v, ta_v, osem0, osem1):
        wid = lax.axis_index("s") * nc + lax.axis_index("c")
        pltpu.sync_copy(im, ids_v.at[pl.ds(0, B * L)])
        pltpu.sync_copy(ic, ids_v.at[pl.ds(B * L, B * L)])
        pltpu.sync_copy(ia, ids_v.at[pl.ds(2 * B * L, B * L)])
        pltpu.sync_copy(pm, tm_v)
        pltpu.sync_copy(pc, tc_v)
        pltpu.sync_copy(pa, ta_v)

        outs = (om, oc, oa)
        tabs = (tm_v, tc_v, ta_v)
        osems = (osem0, osem1)
        for s in range(3):
            m = _MAXP[s]
            tab_v = tabs[s]
            out_ref = outs[s]

            def row_body(r, _, s=s, m=m, tab_v=tab_v, out_ref=out_ref):
                rr = wid + nw * r
                b = rr // L
                i = rr % L
                ib = s * (B * L) + b * L
                idv = plsc.load_gather(
                    ids_v, [jnp.full((16,), ib + i, jnp.int32)])
                for c in range(L // 16):
                    v = ids_v[pl.ds(ib + c * 16, 16)]
                    pos_v[pl.ds(c * 16, 16)] = (
                        (jnp.clip(idv - v, -m, m) + m) * D)
                for h in (0, 1):
                    def _wait(h=h):
                        pltpu.make_async_copy(
                            stage_v.at[h], out_ref.at[0, 0, pl.ds(0, 4)],
                            osems[h]).wait()
                    if s == 0:
                        pl.when(r > 0)(_wait)
                    else:
                        _wait()
                    def c_body(c, _, h=h, tab_v=tab_v):
                        jg = c // 8
                        t16 = (c % 8) * 16
                        p64 = pos_v[pl.ds(c * 16, 16)]
                        for dgl in range(4):
                            for dd in range(8):
                                dabs = 8 * (4 * h + dgl) + dd
                                vals = plsc.load_gather(tab_v, [p64 + dabs])
                                stage_v[h, dgl, jg, dd,
                                        pl.ds(t16, 16)] = vals
                        return 0

                    lax.fori_loop(0, L // 16, c_body, 0)
                    pltpu.async_copy(
                        stage_v.at[h], out_ref.at[b, i, pl.ds(4 * h, 4)],
                        osems[h])
                return 0

            lax.fori_loop(0, rows_per_phase, row_body, 0)

        for x in (0, 1):
            pltpu.make_async_copy(
                stage_v.at[x], om.at[0, 0, pl.ds(0, 4)], osems[x]).wait()

    return sc


_sc_kernel = _make_sc_kernel()


def kernel(melody, chord_ids, annotation_1,
           emb_melody, W_melody, b_melody,
           emb_chord, W_chord, b_chord,
           emb_ann, W_ann, b_ann):
    em = jnp.pad(emb_melody, ((0, _RPAD[0] - _ROWS[0]), (0, 0)))
    ec = jnp.pad(emb_chord, ((0, _RPAD[1] - _ROWS[1]), (0, 0)))
    ea = jnp.pad(emb_ann, ((0, _RPAD[2] - _ROWS[2]), (0, 0)))
    pm, pc, pa = pl.pallas_call(
        _proj_body,
        out_shape=[jax.ShapeDtypeStruct((_RPAD[0], D), jnp.float32),
                   jax.ShapeDtypeStruct((_RPAD[1], D), jnp.float32),
                   jax.ShapeDtypeStruct((_RPAD[2], D), jnp.float32)],
    )(em, W_melody, b_melody.reshape(1, D),
      ec, W_chord, b_chord.reshape(1, D),
      ea, W_ann, b_ann.reshape(1, D))

    im = melody.reshape(B * L).astype(jnp.int32)
    ic = chord_ids.reshape(B * L).astype(jnp.int32)
    ia = annotation_1.reshape(B * L).astype(jnp.int32)
    om, oc, oa = _sc_kernel(pm.reshape(-1), pc.reshape(-1), pa.reshape(-1),
                            im, ic, ia)

    def _fin(o):
        return o.transpose(0, 1, 3, 5, 2, 4).reshape(B, L, L, D)

    return (_fin(om), _fin(oc), _fin(oa))


# speedup vs baseline: 2.6586x; 2.6586x over previous
import functools

import jax
import jax.numpy as jnp
from jax import lax
from jax.experimental import pallas as pl
from jax.experimental.pallas import tpu as pltpu
from jax.experimental.pallas import tpu_sc as plsc

B, L, D = 2, 512, 64
_MAXP = (128, 395, 52)
_ROWS = (257, 791, 105)
_RPAD = (264, 792, 112)


def _proj_body(em, wm, bm, ec, wc, bc, ea, wa, ba, om, oc, oa):
    dn = (((1,), (1,)), ((), ()))
    om[...] = lax.dot_general(wm[...], em[...], dn,
                              preferred_element_type=jnp.float32) + bm[...]
    oc[...] = lax.dot_general(wc[...], ec[...], dn,
                              preferred_element_type=jnp.float32) + bc[...]
    oa[...] = lax.dot_general(wa[...], ea[...], dn,
                              preferred_element_type=jnp.float32) + ba[...]


def _make_sc_kernel():
    info = plsc.get_sparse_core_info()
    nc, ns = info.num_cores, info.num_subcores
    nw = nc * ns
    rows_per_phase = (B * L) // nw
    mesh = plsc.VectorSubcoreMesh(core_axis_name="c", subcore_axis_name="s")

    out_type = [jax.ShapeDtypeStruct((B, L, 8, 4, 8, 128), jnp.float32)
                for _ in range(3)]
    scratch = [
        pltpu.VMEM((3 * B * L,), jnp.int32),
        pltpu.VMEM((L,), jnp.int32),
        pltpu.VMEM((2, 4, 4, 8, 128), jnp.float32),
        pltpu.VMEM((_RPAD[0] * D,), jnp.float32),
        pltpu.VMEM((_RPAD[1] * D,), jnp.float32),
        pltpu.VMEM((_RPAD[2] * D,), jnp.float32),
        pltpu.SemaphoreType.DMA,
        pltpu.SemaphoreType.DMA,
    ]

    @functools.partial(
        pl.kernel, mesh=mesh, out_type=out_type, scratch_types=scratch,
        compiler_params=pltpu.CompilerParams(needs_layout_passes=False,
                                             use_tc_tiling_on_sc=False))
    def sc(pm, pc, pa, im, ic, ia, om, oc, oa,
           ids_v, pos_v, stage_v, tm_v, tc_v, ta_v, osem0, osem1):
        wid = lax.axis_index("s") * nc + lax.axis_index("c")
        pltpu.sync_copy(im, ids_v.at[pl.ds(0, B * L)])
        pltpu.sync_copy(ic, ids_v.at[pl.ds(B * L, B * L)])
        pltpu.sync_copy(ia, ids_v.at[pl.ds(2 * B * L, B * L)])
        pltpu.sync_copy(pm, tm_v)
        pltpu.sync_copy(pc, tc_v)
        pltpu.sync_copy(pa, ta_v)

        outs = (om, oc, oa)
        tabs = (tm_v, tc_v, ta_v)
        osems = (osem0, osem1)
        for s in range(3):
            m = _MAXP[s]
            tab_v = tabs[s]
            out_ref = outs[s]

            def row_body(r, _, s=s, m=m, tab_v=tab_v, out_ref=out_ref):
                rr = wid + nw * r
                b = rr // L
                i = rr % L
                ib = s * (B * L) + b * L
                idv = plsc.load_gather(
                    ids_v, [jnp.full((16,), ib + i, jnp.int32)])
                for c in range(L // 16):
                    v = ids_v[pl.ds(ib + c * 16, 16)]
                    pos_v[pl.ds(c * 16, 16)] = jnp.clip(idv - v, -m, m) + m
                for h in (0, 1):
                    def _wait(h=h):
                        pltpu.make_async_copy(
                            stage_v.at[h], out_ref.at[0, 0, pl.ds(0, 4)],
                            osems[h]).wait()
                    if s == 0:
                        pl.when(r > 0)(_wait)
                    else:
                        _wait()
                    def c_body(c, _, h=h, tab_v=tab_v, rp=_RPAD[s]):
                        jg = c // 8
                        t16 = (c % 8) * 16
                        p = pos_v[pl.ds(c * 16, 16)]
                        for dgl in range(4):
                            for dd in range(8):
                                dabs = 8 * (4 * h + dgl) + dd
                                vals = plsc.load_gather(tab_v,
                                                        [p + dabs * rp])
                                stage_v[h, dgl, jg, dd,
                                        pl.ds(t16, 16)] = vals
                        return 0

                    lax.fori_loop(0, L // 16, c_body, 0)
                    pltpu.async_copy(
                        stage_v.at[h], out_ref.at[b, i, pl.ds(4 * h, 4)],
                        osems[h])
                return 0

            lax.fori_loop(0, rows_per_phase, row_body, 0)

        for x in (0, 1):
            pltpu.make_async_copy(
                stage_v.at[x], om.at[0, 0, pl.ds(0, 4)], osems[x]).wait()

    return sc


_sc_kernel = _make_sc_kernel()


def kernel(melody, chord_ids, annotation_1,
           emb_melody, W_melody, b_melody,
           emb_chord, W_chord, b_chord,
           emb_ann, W_ann, b_ann):
    em = jnp.pad(emb_melody, ((0, _RPAD[0] - _ROWS[0]), (0, 0)))
    ec = jnp.pad(emb_chord, ((0, _RPAD[1] - _ROWS[1]), (0, 0)))
    ea = jnp.pad(emb_ann, ((0, _RPAD[2] - _ROWS[2]), (0, 0)))
    pm, pc, pa = pl.pallas_call(
        _proj_body,
        out_shape=[jax.ShapeDtypeStruct((D, _RPAD[0]), jnp.float32),
                   jax.ShapeDtypeStruct((D, _RPAD[1]), jnp.float32),
                   jax.ShapeDtypeStruct((D, _RPAD[2]), jnp.float32)],
    )(em, W_melody, b_melody.reshape(D, 1),
      ec, W_chord, b_chord.reshape(D, 1),
      ea, W_ann, b_ann.reshape(D, 1))

    im = melody.reshape(B * L).astype(jnp.int32)
    ic = chord_ids.reshape(B * L).astype(jnp.int32)
    ia = annotation_1.reshape(B * L).astype(jnp.int32)
    om, oc, oa = _sc_kernel(pm.reshape(-1), pc.reshape(-1), pa.reshape(-1),
                            im, ic, ia)

    def _fin(o):
        return o.transpose(0, 1, 3, 5, 2, 4).reshape(B, L, L, D)

    return (_fin(om), _fin(oc), _fin(oa))


# speedup vs baseline: 7.5070x; 2.8237x over previous
import functools

import jax
import jax.numpy as jnp
from jax import lax
from jax.experimental import pallas as pl
from jax.experimental.pallas import tpu as pltpu
from jax.experimental.pallas import tpu_sc as plsc

B, L, D = 2, 512, 64
_MAXP = (128, 395, 52)
_ROWS = (257, 791, 105)
_RPAD = (264, 792, 112)


def _proj_body(em, wm, bm, ec, wc, bc, ea, wa, ba, om, oc, oa):
    dn = (((1,), (1,)), ((), ()))
    om[...] = lax.dot_general(wm[...], em[...], dn,
                              preferred_element_type=jnp.float32) + bm[...]
    oc[...] = lax.dot_general(wc[...], ec[...], dn,
                              preferred_element_type=jnp.float32) + bc[...]
    oa[...] = lax.dot_general(wa[...], ea[...], dn,
                              preferred_element_type=jnp.float32) + ba[...]


def _make_sc_kernel():
    info = plsc.get_sparse_core_info()
    nc, ns = info.num_cores, info.num_subcores
    nw = nc * ns
    rows_per_phase = (B * L) // nw
    mesh = plsc.VectorSubcoreMesh(core_axis_name="c", subcore_axis_name="s")

    out_type = [jax.ShapeDtypeStruct((B, L, 8, 4, 8, 128), jnp.float32)
                for _ in range(3)]
    scratch = [
        pltpu.VMEM((3 * B * L,), jnp.int32),
        pltpu.VMEM((L,), jnp.int32),
        pltpu.VMEM((2, 4, 4, 8, 128), jnp.float32),
        pltpu.VMEM((_RPAD[0] * D,), jnp.float32),
        pltpu.VMEM((_RPAD[1] * D,), jnp.float32),
        pltpu.VMEM((_RPAD[2] * D,), jnp.float32),
        pltpu.SemaphoreType.DMA,
        pltpu.SemaphoreType.DMA,
    ]

    @functools.partial(
        pl.kernel, mesh=mesh, out_type=out_type, scratch_types=scratch,
        compiler_params=pltpu.CompilerParams(needs_layout_passes=False,
                                             use_tc_tiling_on_sc=False))
    def sc(pm, pc, pa, im, ic, ia, om, oc, oa,
           ids_v, pos_v, stage_v, tm_v, tc_v, ta_v, osem0, osem1):
        wid = lax.axis_index("s") * nc + lax.axis_index("c")
        pltpu.sync_copy(im, ids_v.at[pl.ds(0, B * L)])
        pltpu.sync_copy(ic, ids_v.at[pl.ds(B * L, B * L)])
        pltpu.sync_copy(ia, ids_v.at[pl.ds(2 * B * L, B * L)])
        pltpu.sync_copy(pm, tm_v)
        pltpu.sync_copy(pc, tc_v)
        pltpu.sync_copy(pa, ta_v)

        outs = (om, oc, oa)
        tabs = (tm_v, tc_v, ta_v)
        osems = (osem0, osem1)
        for s in range(3):
            m = _MAXP[s]
            tab_v = tabs[s]
            out_ref = outs[s]

            def row_body(r, _, s=s, m=m, tab_v=tab_v, out_ref=out_ref):
                rr = wid + nw * r
                b = rr // L
                i = rr % L
                ib = s * (B * L) + b * L
                idv = plsc.load_gather(
                    ids_v, [jnp.full((16,), ib + i, jnp.int32)])
                for c in range(L // 16):
                    v = ids_v[pl.ds(ib + c * 16, 16)]
                    pos_v[pl.ds(c * 16, 16)] = jnp.clip(idv - v, -m, m) + m
                for h in (0, 1):
                    def _wait(h=h):
                        pltpu.make_async_copy(
                            stage_v.at[h], out_ref.at[0, 0, pl.ds(0, 4)],
                            osems[h]).wait()
                    if s == 0:
                        pl.when(r > 0)(_wait)
                    else:
                        _wait()
                    def c_body(c, _, h=h, tab_v=tab_v, rp=_RPAD[s]):
                        jg = c // 8
                        t16 = (c % 8) * 16
                        p = pos_v[pl.ds(c * 16, 16)]
                        for dgl in range(4):
                            vals = [
                                plsc.load_gather(
                                    tab_v, [p + (8 * (4 * h + dgl) + dd) * rp])
                                for dd in range(8)
                            ]
                            for dd in range(8):
                                stage_v[h, dgl, jg, dd,
                                        pl.ds(t16, 16)] = vals[dd]
                        return 0

                    lax.fori_loop(0, L // 16, c_body, 0)
                    pltpu.async_copy(
                        stage_v.at[h], out_ref.at[b, i, pl.ds(4 * h, 4)],
                        osems[h])
                return 0

            lax.fori_loop(0, rows_per_phase, row_body, 0)

        for x in (0, 1):
            pltpu.make_async_copy(
                stage_v.at[x], om.at[0, 0, pl.ds(0, 4)], osems[x]).wait()

    return sc


_sc_kernel = _make_sc_kernel()


def kernel(melody, chord_ids, annotation_1,
           emb_melody, W_melody, b_melody,
           emb_chord, W_chord, b_chord,
           emb_ann, W_ann, b_ann):
    em = jnp.pad(emb_melody, ((0, _RPAD[0] - _ROWS[0]), (0, 0)))
    ec = jnp.pad(emb_chord, ((0, _RPAD[1] - _ROWS[1]), (0, 0)))
    ea = jnp.pad(emb_ann, ((0, _RPAD[2] - _ROWS[2]), (0, 0)))
    pm, pc, pa = pl.pallas_call(
        _proj_body,
        out_shape=[jax.ShapeDtypeStruct((D, _RPAD[0]), jnp.float32),
                   jax.ShapeDtypeStruct((D, _RPAD[1]), jnp.float32),
                   jax.ShapeDtypeStruct((D, _RPAD[2]), jnp.float32)],
    )(em, W_melody, b_melody.reshape(D, 1),
      ec, W_chord, b_chord.reshape(D, 1),
      ea, W_ann, b_ann.reshape(D, 1))

    im = melody.reshape(B * L).astype(jnp.int32)
    ic = chord_ids.reshape(B * L).astype(jnp.int32)
    ia = annotation_1.reshape(B * L).astype(jnp.int32)
    om, oc, oa = _sc_kernel(pm.reshape(-1), pc.reshape(-1), pa.reshape(-1),
                            im, ic, ia)

    def _fin(o):
        return o.transpose(0, 1, 3, 5, 2, 4).reshape(B, L, L, D)

    return (_fin(om), _fin(oc), _fin(oa))
